# initial kernel scaffold (unmeasured)
import jax
import jax.numpy as jnp
from jax import lax
from jax.experimental import pallas as pl
from jax.experimental.pallas import tpu as pltpu

N_Y = 4
M, D = 8192, 2048
CH = M // N_Y
TILE = 512


def _allreduce_body(x_ref, work_ref, recv_ref, a_t, b_t, lsem, send_sems, recv_sems):
    my_x = lax.axis_index("x")
    my_y = lax.axis_index("y")
    my_z = lax.axis_index("z")
    right = lax.rem(my_y + 1, N_Y)
    left = lax.rem(my_y + N_Y - 1, N_Y)

    barrier = pltpu.get_barrier_semaphore()
    for nbr in (left, right):
        pl.semaphore_signal(
            barrier, inc=1, device_id=(my_x, nbr, my_z),
            device_id_type=pl.DeviceIdType.MESH,
        )
    pl.semaphore_wait(barrier, 2)

    cp = pltpu.make_async_copy(x_ref, work_ref, lsem.at[0])
    cp.start()
    cp.wait()

    for s in range(N_Y - 1):
        send_c = lax.rem(my_y - s + 2 * N_Y, N_Y)
        recv_c = lax.rem(my_y - s - 1 + 2 * N_Y, N_Y)
        rdma = pltpu.make_async_remote_copy(
            src_ref=work_ref.at[pl.ds(send_c * CH, CH), :],
            dst_ref=recv_ref.at[s],
            send_sem=send_sems.at[s],
            recv_sem=recv_sems.at[s],
            device_id=(my_x, right, my_z),
            device_id_type=pl.DeviceIdType.MESH,
        )
        rdma.start()
        rdma.wait()
        for t in range(CH // TILE):
            row = recv_c * CH + t * TILE
            ca = pltpu.make_async_copy(
                work_ref.at[pl.ds(row, TILE), :], a_t, lsem.at[0])
            cb = pltpu.make_async_copy(
                recv_ref.at[s, pl.ds(t * TILE, TILE), :], b_t, lsem.at[1])
            ca.start()
            cb.start()
            ca.wait()
            cb.wait()
            a_t[...] = a_t[...] + b_t[...]
            cw = pltpu.make_async_copy(
                a_t, work_ref.at[pl.ds(row, TILE), :], lsem.at[0])
            cw.start()
            cw.wait()

    for t in range(N_Y - 1):
        d = lax.rem(my_y + 1 - t + 2 * N_Y, N_Y)
        rdma = pltpu.make_async_remote_copy(
            src_ref=work_ref.at[pl.ds(d * CH, CH), :],
            dst_ref=work_ref.at[pl.ds(d * CH, CH), :],
            send_sem=send_sems.at[3 + t],
            recv_sem=recv_sems.at[3 + t],
            device_id=(my_x, right, my_z),
            device_id_type=pl.DeviceIdType.MESH,
        )
        rdma.start()
        rdma.wait()


def kernel(partial, resid, gamma):
    xb = partial[0].astype(jnp.bfloat16)
    ysum = pl.pallas_call(
        _allreduce_body,
        out_shape=jax.ShapeDtypeStruct((M, D), jnp.bfloat16),
        in_specs=[pl.BlockSpec(memory_space=pltpu.MemorySpace.HBM)],
        out_specs=pl.BlockSpec(memory_space=pltpu.MemorySpace.HBM),
        scratch_shapes=[
            pltpu.MemorySpace.HBM((N_Y - 1, CH, D), jnp.bfloat16),
            pltpu.MemorySpace.VMEM((TILE, D), jnp.bfloat16),
            pltpu.MemorySpace.VMEM((TILE, D), jnp.bfloat16),
            pltpu.SemaphoreType.DMA((2,)),
            pltpu.SemaphoreType.DMA((6,)),
            pltpu.SemaphoreType.DMA((6,)),
        ],
        compiler_params=pltpu.CompilerParams(collective_id=0),
    )(xb)
    y = ysum.astype(jnp.float32) + resid
    rms = jnp.sqrt(jnp.mean(y * y, axis=-1, keepdims=True) + 1e-6)
    return (y / rms * gamma).astype(jnp.float32)


# baseline (device time: 1745777 ns/iter reference)
import jax
import jax.numpy as jnp
from jax import lax
from jax.experimental import pallas as pl
from jax.experimental.pallas import tpu as pltpu

N_Y = 4
M, D = 8192, 2048
CH = M // N_Y
TILE = 512


def _allreduce_body(x_ref, work_ref, recv_ref, a_t, b_t, lsem, send_sems, recv_sems):
    my_x = lax.axis_index("x")
    my_y = lax.axis_index("y")
    my_z = lax.axis_index("z")
    right = lax.rem(my_y + 1, N_Y)
    left = lax.rem(my_y + N_Y - 1, N_Y)

    barrier = pltpu.get_barrier_semaphore()
    for nbr in (left, right):
        pl.semaphore_signal(
            barrier, inc=1, device_id=(my_x, nbr, my_z),
            device_id_type=pl.DeviceIdType.MESH,
        )
    pl.semaphore_wait(barrier, 2)

    cp = pltpu.make_async_copy(x_ref, work_ref, lsem.at[0])
    cp.start()
    cp.wait()

    for s in range(N_Y - 1):
        send_c = lax.rem(my_y - s + 2 * N_Y, N_Y)
        recv_c = lax.rem(my_y - s - 1 + 2 * N_Y, N_Y)
        rdma = pltpu.make_async_remote_copy(
            src_ref=work_ref.at[pl.ds(send_c * CH, CH), :],
            dst_ref=recv_ref.at[s],
            send_sem=send_sems.at[s],
            recv_sem=recv_sems.at[s],
            device_id=(my_x, right, my_z),
            device_id_type=pl.DeviceIdType.MESH,
        )
        rdma.start()
        rdma.wait()
        for t in range(CH // TILE):
            row = recv_c * CH + t * TILE
            ca = pltpu.make_async_copy(
                work_ref.at[pl.ds(row, TILE), :], a_t, lsem.at[0])
            cb = pltpu.make_async_copy(
                recv_ref.at[s, pl.ds(t * TILE, TILE), :], b_t, lsem.at[1])
            ca.start()
            cb.start()
            ca.wait()
            cb.wait()
            a_t[...] = a_t[...] + b_t[...]
            cw = pltpu.make_async_copy(
                a_t, work_ref.at[pl.ds(row, TILE), :], lsem.at[0])
            cw.start()
            cw.wait()

    for t in range(N_Y - 1):
        d = lax.rem(my_y + 1 - t + 2 * N_Y, N_Y)
        rdma = pltpu.make_async_remote_copy(
            src_ref=work_ref.at[pl.ds(d * CH, CH), :],
            dst_ref=work_ref.at[pl.ds(d * CH, CH), :],
            send_sem=send_sems.at[3 + t],
            recv_sem=recv_sems.at[3 + t],
            device_id=(my_x, right, my_z),
            device_id_type=pl.DeviceIdType.MESH,
        )
        rdma.start()
        rdma.wait()


def kernel(partial, resid, gamma):
    xb = partial[0].astype(jnp.bfloat16)
    ysum, _ = pl.pallas_call(
        _allreduce_body,
        out_shape=[
            jax.ShapeDtypeStruct((M, D), jnp.bfloat16),
            jax.ShapeDtypeStruct((N_Y - 1, CH, D), jnp.bfloat16),
        ],
        in_specs=[pl.BlockSpec(memory_space=pltpu.MemorySpace.HBM)],
        out_specs=[
            pl.BlockSpec(memory_space=pltpu.MemorySpace.HBM),
            pl.BlockSpec(memory_space=pltpu.MemorySpace.HBM),
        ],
        scratch_shapes=[
            pltpu.MemorySpace.VMEM((TILE, D), jnp.bfloat16),
            pltpu.MemorySpace.VMEM((TILE, D), jnp.bfloat16),
            pltpu.SemaphoreType.DMA((2,)),
            pltpu.SemaphoreType.DMA((6,)),
            pltpu.SemaphoreType.DMA((6,)),
        ],
        compiler_params=pltpu.CompilerParams(collective_id=0),
    )(xb)
    y = ysum.astype(jnp.float32) + resid
    rms = jnp.sqrt(jnp.mean(y * y, axis=-1, keepdims=True) + 1e-6)
    return (y / rms * gamma).astype(jnp.float32)


# device time: 365930 ns/iter; 4.7708x vs baseline; 4.7708x over previous
import jax
import jax.numpy as jnp
from jax import lax
from jax.experimental import pallas as pl
from jax.experimental.pallas import tpu as pltpu

N_Y = 4
N_Z = 4
M, D = 8192, 2048
CH = M // N_Z
HBR = CH // 2
QR = HBR // N_Y

_MESH = pl.DeviceIdType.MESH


def _body(x_ref, work_ref, recvq, sbuf, xt,
          p1_s, p1_r, ag_s, ag_r, zr_s, zr_r, zl_s, zl_r, xs_s, xs_r, lsem):
    my_x = lax.axis_index("x")
    my_y = lax.axis_index("y")
    my_z = lax.axis_index("z")
    m4 = lambda v: lax.rem(v + 8, 4)
    right_y = m4(my_y + 1)
    left_y = m4(my_y - 1)
    R0 = my_z * CH + my_x * HBR
    rows_mine = lambda g: g * CH + my_x * HBR
    rows_other = lambda g: g * CH + (1 - my_x) * HBR

    barrier = pltpu.get_barrier_semaphore()

    def sig(dev):
        pl.semaphore_signal(barrier, inc=1, device_id=dev,
                            device_id_type=_MESH)

    sig((my_x, left_y, my_z))
    sig((my_x, right_y, my_z))
    sig((1 - my_x, my_y, my_z))

    @pl.when(my_z > 0)
    def _():
        sig((my_x, my_y, my_z - 1))

    @pl.when(my_z < N_Z - 1)
    def _():
        sig((my_x, my_y, my_z + 1))

    z_edge = jnp.logical_or(my_z == 0, my_z == N_Z - 1)

    @pl.when(z_edge)
    def _():
        pl.semaphore_wait(barrier, 4)

    @pl.when(jnp.logical_not(z_edge))
    def _():
        pl.semaphore_wait(barrier, 5)

    c0 = pltpu.make_async_copy(
        x_ref.at[0, pl.ds(R0 + m4(my_y) * QR, QR), :], xt, lsem.at[0])
    c0.start()
    c0.wait()
    sbuf[0, :, :] = xt[:, :].astype(jnp.bfloat16)

    for s in range(N_Y - 1):
        recv_q = m4(my_y - s - 1)
        rdma = pltpu.make_async_remote_copy(
            src_ref=sbuf.at[s % 2],
            dst_ref=recvq.at[s],
            send_sem=p1_s.at[s],
            recv_sem=p1_r.at[s],
            device_id=(my_x, right_y, my_z),
            device_id_type=_MESH,
        )
        rdma.start()
        cx = pltpu.make_async_copy(
            x_ref.at[0, pl.ds(R0 + recv_q * QR, QR), :], xt, lsem.at[0])
        cx.start()
        rdma.wait()
        cx.wait()
        sbuf[(s + 1) % 2, :, :] = xt[:, :].astype(jnp.bfloat16) + recvq[s, :, :]

    own_q = m4(my_y + 1)
    cw = pltpu.make_async_copy(
        sbuf.at[1], work_ref.at[pl.ds(R0 + own_q * QR, QR), :], lsem.at[1])
    cw.start()
    for t in range(N_Y - 1):
        d = m4(my_y + 1 - t)
        src = sbuf.at[1] if t == 0 else work_ref.at[pl.ds(R0 + d * QR, QR), :]
        rdma = pltpu.make_async_remote_copy(
            src_ref=src,
            dst_ref=work_ref.at[pl.ds(R0 + d * QR, QR), :],
            send_sem=ag_s.at[t],
            recv_sem=ag_r.at[t],
            device_id=(my_x, right_y, my_z),
            device_id_type=_MESH,
        )
        rdma.start()
        rdma.wait()
    cw.wait()

    def x_fwd(g, slot):
        return pltpu.make_async_remote_copy(
            src_ref=work_ref.at[pl.ds(rows_mine(g), HBR), :],
            dst_ref=work_ref.at[pl.ds(rows_mine(g), HBR), :],
            send_sem=xs_s.at[slot],
            recv_sem=xs_r.at[slot],
            device_id=(1 - my_x, my_y, my_z),
            device_id_type=_MESH,
        )

    def z_copy(g, dz, send_sem, recv_sem):
        return pltpu.make_async_remote_copy(
            src_ref=work_ref.at[pl.ds(rows_mine(g), HBR), :],
            dst_ref=work_ref.at[pl.ds(rows_mine(g), HBR), :],
            send_sem=send_sem,
            recv_sem=recv_sem,
            device_id=(my_x, my_y, my_z + dz),
            device_id_type=_MESH,
        )

    x_fwd(my_z, 0).start()

    for t in range(N_Z - 1):
        @pl.when(jnp.logical_and(my_z >= t, my_z < N_Z - 1))
        def _():
            z_copy(my_z - t, +1, zr_s.at[t], zr_r.at[t]).start()

        @pl.when(jnp.logical_and(my_z <= N_Z - 1 - t, my_z > 0))
        def _():
            z_copy(my_z + t, -1, zl_s.at[t], zl_r.at[t]).start()

        @pl.when(my_z >= t + 1)
        def _():
            gl = my_z - 1 - t
            z_copy(gl, +1, zr_s.at[t], zr_r.at[t]).wait_recv()
            x_fwd(gl, 1 + 2 * t).start()

        @pl.when(my_z <= N_Z - 2 - t)
        def _():
            gr = my_z + 1 + t
            z_copy(gr, -1, zl_s.at[t], zl_r.at[t]).wait_recv()
            x_fwd(gr, 2 + 2 * t).start()

    def x_recv_wait(g, slot):
        pltpu.make_async_remote_copy(
            src_ref=work_ref.at[pl.ds(rows_other(g), HBR), :],
            dst_ref=work_ref.at[pl.ds(rows_other(g), HBR), :],
            send_sem=xs_s.at[slot],
            recv_sem=xs_r.at[slot],
            device_id=(1 - my_x, my_y, my_z),
            device_id_type=_MESH,
        ).wait_recv()

    x_recv_wait(my_z, 0)
    x_fwd(my_z, 0).wait_send()
    for t in range(N_Z - 1):
        @pl.when(my_z >= t + 1)
        def _():
            gl = my_z - 1 - t
            x_recv_wait(gl, 1 + 2 * t)
            x_fwd(gl, 1 + 2 * t).wait_send()

        @pl.when(my_z <= N_Z - 2 - t)
        def _():
            gr = my_z + 1 + t
            x_recv_wait(gr, 2 + 2 * t)
            x_fwd(gr, 2 + 2 * t).wait_send()

        @pl.when(jnp.logical_and(my_z >= t, my_z < N_Z - 1))
        def _():
            z_copy(my_z - t, +1, zr_s.at[t], zr_r.at[t]).wait_send()

        @pl.when(jnp.logical_and(my_z <= N_Z - 1 - t, my_z > 0))
        def _():
            z_copy(my_z + t, -1, zl_s.at[t], zl_r.at[t]).wait_send()


def kernel(partial, resid, gamma):
    ysum = pl.pallas_call(
        _body,
        out_shape=jax.ShapeDtypeStruct((M, D), jnp.bfloat16),
        in_specs=[pl.BlockSpec(memory_space=pltpu.MemorySpace.HBM)],
        out_specs=pl.BlockSpec(memory_space=pltpu.MemorySpace.HBM),
        scratch_shapes=[
            pltpu.MemorySpace.VMEM((N_Y - 1, QR, D), jnp.bfloat16),
            pltpu.MemorySpace.VMEM((2, QR, D), jnp.bfloat16),
            pltpu.MemorySpace.VMEM((QR, D), jnp.float32),
            pltpu.SemaphoreType.DMA((3,)),
            pltpu.SemaphoreType.DMA((3,)),
            pltpu.SemaphoreType.DMA((3,)),
            pltpu.SemaphoreType.DMA((3,)),
            pltpu.SemaphoreType.DMA((3,)),
            pltpu.SemaphoreType.DMA((3,)),
            pltpu.SemaphoreType.DMA((3,)),
            pltpu.SemaphoreType.DMA((3,)),
            pltpu.SemaphoreType.DMA((7,)),
            pltpu.SemaphoreType.DMA((7,)),
            pltpu.SemaphoreType.DMA((2,)),
        ],
        compiler_params=pltpu.CompilerParams(collective_id=0),
    )(partial)
    y = ysum.astype(jnp.float32) + resid
    rms = jnp.sqrt(jnp.mean(y * y, axis=-1, keepdims=True) + 1e-6)
    return (y / rms * gamma).astype(jnp.float32)


# device time: 354046 ns/iter; 4.9309x vs baseline; 1.0336x over previous
import jax
import jax.numpy as jnp
from jax import lax
from jax.experimental import pallas as pl
from jax.experimental.pallas import tpu as pltpu

N_Y = 4
N_Z = 4
M, D = 8192, 2048
CH = M // N_Z
HBR = CH // 2
QR = HBR // N_Y
NT = HBR // QR

_MESH = pl.DeviceIdType.MESH


def _body(x_ref, resid_ref, gamma_ref, out_ref, work_ref, recvq, sbuf, xt,
          nb_t, nr_t, ot,
          p1_s, p1_r, ag_s, ag_r, zr_s, zr_r, zl_s, zl_r, xs_s, xs_r,
          lsem, nsem, osem):
    my_x = lax.axis_index("x")
    my_y = lax.axis_index("y")
    my_z = lax.axis_index("z")
    m4 = lambda v: lax.rem(v + 8, 4)
    right_y = m4(my_y + 1)
    left_y = m4(my_y - 1)
    R0 = my_z * CH + my_x * HBR
    rows_mine = lambda g: g * CH + my_x * HBR
    rows_other = lambda g: g * CH + (1 - my_x) * HBR

    barrier = pltpu.get_barrier_semaphore()

    def sig(dev):
        pl.semaphore_signal(barrier, inc=1, device_id=dev,
                            device_id_type=_MESH)

    sig((my_x, left_y, my_z))
    sig((my_x, right_y, my_z))
    sig((1 - my_x, my_y, my_z))

    @pl.when(my_z > 0)
    def _():
        sig((my_x, my_y, my_z - 1))

    @pl.when(my_z < N_Z - 1)
    def _():
        sig((my_x, my_y, my_z + 1))

    z_edge = jnp.logical_or(my_z == 0, my_z == N_Z - 1)

    @pl.when(z_edge)
    def _():
        pl.semaphore_wait(barrier, 4)

    @pl.when(jnp.logical_not(z_edge))
    def _():
        pl.semaphore_wait(barrier, 5)

    c0 = pltpu.make_async_copy(
        x_ref.at[0, pl.ds(R0 + m4(my_y) * QR, QR), :], xt, lsem.at[0])
    c0.start()
    c0.wait()
    sbuf[0, :, :] = xt[:, :].astype(jnp.bfloat16)

    for s in range(N_Y - 1):
        recv_q = m4(my_y - s - 1)
        rdma = pltpu.make_async_remote_copy(
            src_ref=sbuf.at[s % 2],
            dst_ref=recvq.at[s],
            send_sem=p1_s.at[s],
            recv_sem=p1_r.at[s],
            device_id=(my_x, right_y, my_z),
            device_id_type=_MESH,
        )
        rdma.start()
        cx = pltpu.make_async_copy(
            x_ref.at[0, pl.ds(R0 + recv_q * QR, QR), :], xt, lsem.at[0])
        cx.start()
        rdma.wait()
        cx.wait()
        sbuf[(s + 1) % 2, :, :] = xt[:, :].astype(jnp.bfloat16) + recvq[s, :, :]

    own_q = m4(my_y + 1)
    cw = pltpu.make_async_copy(
        sbuf.at[1], work_ref.at[pl.ds(R0 + own_q * QR, QR), :], lsem.at[1])
    cw.start()
    for t in range(N_Y - 1):
        d = m4(my_y + 1 - t)
        src = sbuf.at[1] if t == 0 else work_ref.at[pl.ds(R0 + d * QR, QR), :]
        rdma = pltpu.make_async_remote_copy(
            src_ref=src,
            dst_ref=work_ref.at[pl.ds(R0 + d * QR, QR), :],
            send_sem=ag_s.at[t],
            recv_sem=ag_r.at[t],
            device_id=(my_x, right_y, my_z),
            device_id_type=_MESH,
        )
        rdma.start()
        rdma.wait()
    cw.wait()

    g2 = gamma_ref[:].reshape(1, D)

    def norm_block(rb):
        def step(i, carry):
            r = rb + i * QR
            cb = pltpu.make_async_copy(
                work_ref.at[pl.ds(r, QR), :], nb_t, nsem.at[0])
            cr = pltpu.make_async_copy(
                resid_ref.at[pl.ds(r, QR), :], nr_t, nsem.at[1])
            cb.start()
            cr.start()
            cb.wait()
            cr.wait()
            y = nb_t[:, :].astype(jnp.float32) + nr_t[:, :]
            rms = jnp.sqrt(jnp.mean(y * y, axis=1, keepdims=True) + 1e-6)
            ot[:, :] = y / rms * g2
            co = pltpu.make_async_copy(ot, out_ref.at[pl.ds(r, QR), :],
                                       osem.at[0])
            co.start()
            co.wait()
            return carry

        lax.fori_loop(0, NT, step, 0)

    def x_fwd(g, slot):
        return pltpu.make_async_remote_copy(
            src_ref=work_ref.at[pl.ds(rows_mine(g), HBR), :],
            dst_ref=work_ref.at[pl.ds(rows_mine(g), HBR), :],
            send_sem=xs_s.at[slot],
            recv_sem=xs_r.at[slot],
            device_id=(1 - my_x, my_y, my_z),
            device_id_type=_MESH,
        )

    def z_copy(g, dz, send_sem, recv_sem):
        return pltpu.make_async_remote_copy(
            src_ref=work_ref.at[pl.ds(rows_mine(g), HBR), :],
            dst_ref=work_ref.at[pl.ds(rows_mine(g), HBR), :],
            send_sem=send_sem,
            recv_sem=recv_sem,
            device_id=(my_x, my_y, my_z + dz),
            device_id_type=_MESH,
        )

    x_fwd(my_z, 0).start()

    for t in range(N_Z - 1):
        @pl.when(jnp.logical_and(my_z >= t, my_z < N_Z - 1))
        def _():
            z_copy(my_z - t, +1, zr_s.at[t], zr_r.at[t]).start()

        @pl.when(jnp.logical_and(my_z <= N_Z - 1 - t, my_z > 0))
        def _():
            z_copy(my_z + t, -1, zl_s.at[t], zl_r.at[t]).start()

        if t == 0:
            norm_block(R0)
        else:
            @pl.when(my_z >= t)
            def _():
                norm_block(rows_mine(my_z - t))

            @pl.when(my_z <= N_Z - 1 - t)
            def _():
                norm_block(rows_mine(my_z + t))

        @pl.when(my_z >= t + 1)
        def _():
            gl = my_z - 1 - t
            z_copy(gl, +1, zr_s.at[t], zr_r.at[t]).wait_recv()
            x_fwd(gl, 1 + 2 * t).start()

        @pl.when(my_z <= N_Z - 2 - t)
        def _():
            gr = my_z + 1 + t
            z_copy(gr, -1, zl_s.at[t], zl_r.at[t]).wait_recv()
            x_fwd(gr, 2 + 2 * t).start()

    def x_recv_wait(g, slot):
        pltpu.make_async_remote_copy(
            src_ref=work_ref.at[pl.ds(rows_other(g), HBR), :],
            dst_ref=work_ref.at[pl.ds(rows_other(g), HBR), :],
            send_sem=xs_s.at[slot],
            recv_sem=xs_r.at[slot],
            device_id=(1 - my_x, my_y, my_z),
            device_id_type=_MESH,
        ).wait_recv()

    @pl.when(my_z >= N_Z - 1)
    def _():
        norm_block(rows_mine(my_z - (N_Z - 1)))

    @pl.when(my_z <= 0)
    def _():
        norm_block(rows_mine(my_z + (N_Z - 1)))

    x_recv_wait(my_z, 0)
    norm_block(rows_other(my_z))
    x_fwd(my_z, 0).wait_send()
    for t in range(N_Z - 1):
        @pl.when(my_z >= t + 1)
        def _():
            gl = my_z - 1 - t
            x_recv_wait(gl, 1 + 2 * t)
            norm_block(rows_other(gl))
            x_fwd(gl, 1 + 2 * t).wait_send()

        @pl.when(my_z <= N_Z - 2 - t)
        def _():
            gr = my_z + 1 + t
            x_recv_wait(gr, 2 + 2 * t)
            norm_block(rows_other(gr))
            x_fwd(gr, 2 + 2 * t).wait_send()

        @pl.when(jnp.logical_and(my_z >= t, my_z < N_Z - 1))
        def _():
            z_copy(my_z - t, +1, zr_s.at[t], zr_r.at[t]).wait_send()

        @pl.when(jnp.logical_and(my_z <= N_Z - 1 - t, my_z > 0))
        def _():
            z_copy(my_z + t, -1, zl_s.at[t], zl_r.at[t]).wait_send()


def kernel(partial, resid, gamma):
    out, _ = pl.pallas_call(
        _body,
        out_shape=[
            jax.ShapeDtypeStruct((M, D), jnp.float32),
            jax.ShapeDtypeStruct((M, D), jnp.bfloat16),
        ],
        in_specs=[
            pl.BlockSpec(memory_space=pltpu.MemorySpace.HBM),
            pl.BlockSpec(memory_space=pltpu.MemorySpace.HBM),
            pl.BlockSpec(memory_space=pltpu.MemorySpace.VMEM),
        ],
        out_specs=[
            pl.BlockSpec(memory_space=pltpu.MemorySpace.HBM),
            pl.BlockSpec(memory_space=pltpu.MemorySpace.HBM),
        ],
        scratch_shapes=[
            pltpu.MemorySpace.VMEM((N_Y - 1, QR, D), jnp.bfloat16),
            pltpu.MemorySpace.VMEM((2, QR, D), jnp.bfloat16),
            pltpu.MemorySpace.VMEM((QR, D), jnp.float32),
            pltpu.MemorySpace.VMEM((QR, D), jnp.bfloat16),
            pltpu.MemorySpace.VMEM((QR, D), jnp.float32),
            pltpu.MemorySpace.VMEM((QR, D), jnp.float32),
            pltpu.SemaphoreType.DMA((3,)),
            pltpu.SemaphoreType.DMA((3,)),
            pltpu.SemaphoreType.DMA((3,)),
            pltpu.SemaphoreType.DMA((3,)),
            pltpu.SemaphoreType.DMA((3,)),
            pltpu.SemaphoreType.DMA((3,)),
            pltpu.SemaphoreType.DMA((3,)),
            pltpu.SemaphoreType.DMA((3,)),
            pltpu.SemaphoreType.DMA((7,)),
            pltpu.SemaphoreType.DMA((7,)),
            pltpu.SemaphoreType.DMA((2,)),
            pltpu.SemaphoreType.DMA((2,)),
            pltpu.SemaphoreType.DMA((1,)),
        ],
        compiler_params=pltpu.CompilerParams(collective_id=0),
    )(partial, resid, gamma)
    return out
